# baseline (speedup 1.0000x reference)
import functools

import jax
import jax.numpy as jnp
from jax import lax
from jax.experimental import pallas as pl
from jax.experimental.pallas import tpu as pltpu
from jax.experimental.pallas import tpu_sc as plsc

N = 10000
K = 16
D_NODE = 256
D_EDGE = 16

L = 16
NC = 2
NS = 16
NW = NC * NS
G = 4
ROWS = G * K
N_PAD = 10240
PER_W = N_PAD // NW
NGRP = PER_W // G
BN = 1000


def _attn_body(nodes_ref, edges_ref, inv_ref, wq_ref, wk_ref, b_ref, pk_ref):
    f32 = jnp.float32
    q = jnp.dot(nodes_ref[...], wq_ref[...], preferred_element_type=f32)
    r = lax.dot_general(q, wk_ref[...], (((1,), (1,)), ((), ())),
                        preferred_element_type=f32)
    r = r * inv_ref[...]
    cmod = lax.broadcasted_iota(jnp.int32, (D_EDGE, K * D_EDGE), 1) % D_EDGE
    drow = lax.broadcasted_iota(jnp.int32, (D_EDGE, K * D_EDGE), 0)
    T = (cmod == drow).astype(f32)
    rt = jnp.dot(r, T, preferred_element_type=f32)
    p = rt * edges_ref[...]
    cdiv = lax.broadcasted_iota(jnp.int32, (K * D_EDGE, K), 0) // D_EDGE
    jcol = lax.broadcasted_iota(jnp.int32, (K * D_EDGE, K), 1)
    S = (cdiv == jcol).astype(f32)
    qd = jnp.dot(p, S, preferred_element_type=f32)
    m = jnp.max(qd, axis=1, keepdims=True)
    ex = jnp.exp(qd - m)
    b_ref[...] = ex / jnp.sum(ex, axis=1, keepdims=True)

    n = nodes_ref[...]

    def rne16(x):
        xb = lax.bitcast_convert_type(x, jnp.int32)
        xb = xb + 32767 + (lax.shift_right_logical(xb, 16) & 1)
        return lax.shift_right_logical(xb, 16)

    lo = rne16(n[:, :D_NODE // 2])
    hi = rne16(n[:, D_NODE // 2:])
    pk_ref[...] = lo | lax.shift_left(hi, 16)


def _attn_weights(nodes, edges2d, inv2d, wq, wk):
    grid = N // BN
    return pl.pallas_call(
        _attn_body,
        grid=(grid,),
        in_specs=[
            pl.BlockSpec((BN, D_NODE), lambda i: (i, 0)),
            pl.BlockSpec((BN, K * D_EDGE), lambda i: (i, 0)),
            pl.BlockSpec((BN, 1), lambda i: (i, 0)),
            pl.BlockSpec((D_NODE, D_EDGE), lambda i: (0, 0)),
            pl.BlockSpec((D_EDGE, D_EDGE), lambda i: (0, 0)),
        ],
        out_specs=[
            pl.BlockSpec((BN, K), lambda i: (i, 0)),
            pl.BlockSpec((BN, D_NODE // 2), lambda i: (i, 0)),
        ],
        out_shape=[
            jax.ShapeDtypeStruct((N, K), jnp.float32),
            jax.ShapeDtypeStruct((N_PAD, D_NODE // 2), jnp.int32),
        ],
    )(nodes, edges2d, inv2d, wq, wk)


NBUF = 4
N_TAB = 10112


def _sc_mix_body(nodes_hbm, nlist_hbm, b_hbm, out_hbm,
                 idx_v, b_v, buf0, buf1, buf2, buf3, ost0, ost1, shared,
                 sg0, sg1, sg2, sg3, so0, so1):
    bufs = (buf0, buf1, buf2, buf3)
    sgs = (sg0, sg1, sg2, sg3)
    osts = (ost0, ost1)
    sos = (so0, so1)

    wid = lax.axis_index("s") * NC + lax.axis_index("c")
    base = wid * PER_W

    s = lax.axis_index("s")
    chunk = N_TAB // NS
    pltpu.sync_copy(nodes_hbm.at[pl.ds(s * chunk, chunk)],
                    shared.at[pl.ds(s * chunk, chunk)])

    pltpu.sync_copy(nlist_hbm.at[wid], idx_v)
    pltpu.sync_copy(b_hbm.at[wid], b_v)
    plsc.subcore_barrier()

    for r in range(NBUF):
        pltpu.async_copy(shared.at[idx_v.at[r]], bufs[r], sgs[r])

    dnums = lax.GatherDimensionNumbers(
        offset_dims=(), collapsed_slice_dims=(0,), start_index_map=(0,))

    def compute(grp, buf, ost):
        def node_body(i, carry):
            node = grp * G + i
            bvec = b_v[pl.ds(node * K, K)]
            bjs = [
                lax.gather(
                    bvec, jnp.full((L, 1), j, jnp.int32), dnums,
                    slice_sizes=(1,),
                    mode=lax.GatherScatterMode.PROMISE_IN_BOUNDS)
                for j in range(K)
            ]
            hi_mask = jnp.full((L,), -65536, jnp.int32)

            def rne(x):
                xb = lax.bitcast_convert_type(x, jnp.int32)
                xb = xb + 32767 + (lax.shift_right_logical(xb, 16) & 1)
                return lax.shift_right_logical(xb, 16)

            for t in range(D_NODE // (2 * L)):
                acc_e = jnp.zeros((L,), jnp.float32)
                acc_o = jnp.zeros((L,), jnp.float32)
                for j in range(K):
                    vi = buf[i * K + j, pl.ds(t * L, L)]
                    ev = lax.bitcast_convert_type(
                        lax.shift_left(vi, 16), jnp.float32)
                    od = lax.bitcast_convert_type(vi & hi_mask, jnp.float32)
                    acc_e = acc_e + bjs[j] * ev
                    acc_o = acc_o + bjs[j] * od
                ost[i, pl.ds(t * L, L)] = rne(acc_e) | lax.shift_left(
                    rne(acc_o), 16)
            return carry
        lax.fori_loop(0, G, node_body, 0)

    def iter_body(it, carry):
        for r in range(NBUF):
            grp = it * NBUF + r
            p = r % 2
            pltpu.make_async_copy(
                shared.at[idx_v.at[grp]], bufs[r], sgs[r]).wait()

            @pl.when(grp >= 2)
            def _():
                pltpu.make_async_copy(
                    osts[p], out_hbm.at[pl.ds(base, G)], sos[p]).wait()

            compute(grp, bufs[r], osts[p])
            pltpu.async_copy(
                osts[p], out_hbm.at[pl.ds(base + grp * G, G)], sos[p])
            nxt = grp + NBUF

            @pl.when(nxt < NGRP)
            def _():
                pltpu.async_copy(shared.at[idx_v.at[nxt]], bufs[r], sgs[r])
        return carry

    lax.fori_loop(0, NGRP // NBUF, iter_body, 0)
    for p in range(2):
        pltpu.make_async_copy(
            osts[p], out_hbm.at[pl.ds(base, G)], sos[p]).wait()


def _sc_mix(nodes, nlist_w, b_w):
    mesh = plsc.VectorSubcoreMesh(core_axis_name="c", subcore_axis_name="s")
    kern = functools.partial(
        pl.kernel,
        mesh=mesh,
        out_type=jax.ShapeDtypeStruct((N_PAD, D_NODE // 2), jnp.int32),
        scratch_types=[
            pltpu.VMEM((NGRP, ROWS), jnp.int32),
            pltpu.VMEM((PER_W * K,), jnp.float32),
        ] + [pltpu.VMEM((ROWS, D_NODE // 2), jnp.int32)] * NBUF
          + [pltpu.VMEM((G, D_NODE // 2), jnp.int32)] * 2
          + [pltpu.VMEM_SHARED((N_TAB, D_NODE // 2), jnp.int32)]
          + [pltpu.SemaphoreType.DMA] * (NBUF + 2),
    )(_sc_mix_body)
    return kern(nodes, nlist_w, b_w)


def _out_body(mixed_ref, wvlo_ref, wvhi_ref, out_ref):
    f32 = jnp.float32
    vi = mixed_ref[...]
    ev = lax.bitcast_convert_type(lax.shift_left(vi, 16), f32)
    od = lax.bitcast_convert_type(vi & jnp.int32(-65536), f32)
    acc = (jnp.dot(ev, wvlo_ref[...], preferred_element_type=f32)
           + jnp.dot(od, wvhi_ref[...], preferred_element_type=f32))
    out_ref[...] = jnp.maximum(acc, 0.0)


def _out_proj(mixed_pk, wv):
    grid = N // BN
    return pl.pallas_call(
        _out_body,
        grid=(grid,),
        in_specs=[
            pl.BlockSpec((BN, D_NODE // 2), lambda i: (i, 0)),
            pl.BlockSpec((D_NODE // 2, D_NODE), lambda i: (0, 0)),
            pl.BlockSpec((D_NODE // 2, D_NODE), lambda i: (0, 0)),
        ],
        out_specs=pl.BlockSpec((BN, D_NODE), lambda i: (i, 0)),
        out_shape=jax.ShapeDtypeStruct((N, D_NODE), jnp.float32),
    )(mixed_pk, wv[:D_NODE // 2], wv[D_NODE // 2:])


def kernel(nodes, nlist, edges, inv_degree, wq, wk, wv):
    edges2d = edges.reshape(N, K * D_EDGE)
    inv2d = inv_degree.reshape(N, 1)

    b, nodes_pk = _attn_weights(nodes, edges2d, inv2d, wq, wk)

    nlist32 = nlist.astype(jnp.int32)
    nlist_pad = jnp.pad(nlist32, ((0, N_PAD - N), (0, 0)))
    nlist_w = nlist_pad.reshape(NW, NGRP, ROWS)
    b_pad = jnp.pad(b, ((0, N_PAD - N), (0, 0)))
    b_w = b_pad.reshape(NW, PER_W * K)

    mixed = _sc_mix(nodes_pk, nlist_w, b_w)
    return _out_proj(mixed, wv)


# speedup vs baseline: 1.0735x; 1.0735x over previous
import functools

import jax
import jax.numpy as jnp
from jax import lax
from jax.experimental import pallas as pl
from jax.experimental.pallas import tpu as pltpu
from jax.experimental.pallas import tpu_sc as plsc

N = 10000
K = 16
D_NODE = 256
D_EDGE = 16

L = 16
NC = 2
NS = 16
NW = NC * NS
G = 4
ROWS = G * K
N_PAD = 10240
PER_W = N_PAD // NW
NGRP = PER_W // G
BN = 2000


def _attn_body(nodes_ref, edges_ref, inv_ref, wq_ref, wk_ref, b_ref, pk_ref):
    f32 = jnp.float32
    q = jnp.dot(nodes_ref[...], wq_ref[...], preferred_element_type=f32)
    r = lax.dot_general(q, wk_ref[...], (((1,), (1,)), ((), ())),
                        preferred_element_type=f32)
    r = r * inv_ref[...]
    cmod = lax.broadcasted_iota(jnp.int32, (D_EDGE, K * D_EDGE), 1) % D_EDGE
    drow = lax.broadcasted_iota(jnp.int32, (D_EDGE, K * D_EDGE), 0)
    T = (cmod == drow).astype(f32)
    rt = jnp.dot(r, T, preferred_element_type=f32)
    p = rt * edges_ref[...]
    cdiv = lax.broadcasted_iota(jnp.int32, (K * D_EDGE, K), 0) // D_EDGE
    jcol = lax.broadcasted_iota(jnp.int32, (K * D_EDGE, K), 1)
    S = (cdiv == jcol).astype(f32)
    qd = jnp.dot(p, S, preferred_element_type=f32)
    m = jnp.max(qd, axis=1, keepdims=True)
    ex = jnp.exp(qd - m)
    b_ref[...] = ex / jnp.sum(ex, axis=1, keepdims=True)

    n = nodes_ref[...]

    def rne16(x):
        xb = lax.bitcast_convert_type(x, jnp.int32)
        xb = xb + 32767 + (lax.shift_right_logical(xb, 16) & 1)
        return lax.shift_right_logical(xb, 16)

    lo = rne16(n[:, :D_NODE // 2])
    hi = rne16(n[:, D_NODE // 2:])
    pk_ref[...] = lo | lax.shift_left(hi, 16)


def _attn_weights(nodes, edges2d, inv2d, wq, wk):
    grid = N // BN
    return pl.pallas_call(
        _attn_body,
        grid=(grid,),
        in_specs=[
            pl.BlockSpec((BN, D_NODE), lambda i: (i, 0)),
            pl.BlockSpec((BN, K * D_EDGE), lambda i: (i, 0)),
            pl.BlockSpec((BN, 1), lambda i: (i, 0)),
            pl.BlockSpec((D_NODE, D_EDGE), lambda i: (0, 0)),
            pl.BlockSpec((D_EDGE, D_EDGE), lambda i: (0, 0)),
        ],
        out_specs=[
            pl.BlockSpec((BN, K), lambda i: (i, 0)),
            pl.BlockSpec((BN, D_NODE // 2), lambda i: (i, 0)),
        ],
        out_shape=[
            jax.ShapeDtypeStruct((N, K), jnp.float32),
            jax.ShapeDtypeStruct((N_PAD, D_NODE // 2), jnp.int32),
        ],
    )(nodes, edges2d, inv2d, wq, wk)


NBUF = 4
N_TAB = 10112


def _sc_mix_body(nodes_hbm, nlist_hbm, b_hbm, out_hbm,
                 idx_v, b_v, buf0, buf1, buf2, buf3, ost0, ost1, shared,
                 sg0, sg1, sg2, sg3, so0, so1):
    bufs = (buf0, buf1, buf2, buf3)
    sgs = (sg0, sg1, sg2, sg3)
    osts = (ost0, ost1)
    sos = (so0, so1)

    wid = lax.axis_index("s") * NC + lax.axis_index("c")
    base = wid * PER_W

    s = lax.axis_index("s")
    chunk = N_TAB // NS
    pltpu.sync_copy(nodes_hbm.at[pl.ds(s * chunk, chunk)],
                    shared.at[pl.ds(s * chunk, chunk)])

    pltpu.sync_copy(nlist_hbm.at[wid], idx_v)
    pltpu.sync_copy(b_hbm.at[wid], b_v)
    plsc.subcore_barrier()

    for r in range(NBUF):
        pltpu.async_copy(shared.at[idx_v.at[r]], bufs[r], sgs[r])

    dnums = lax.GatherDimensionNumbers(
        offset_dims=(), collapsed_slice_dims=(0,), start_index_map=(0,))

    def compute(grp, buf, ost):
        def node_body(i, carry):
            node = grp * G + i
            bvec = b_v[pl.ds(node * K, K)]
            bjs = [
                lax.gather(
                    bvec, jnp.full((L, 1), j, jnp.int32), dnums,
                    slice_sizes=(1,),
                    mode=lax.GatherScatterMode.PROMISE_IN_BOUNDS)
                for j in range(K)
            ]
            hi_mask = jnp.full((L,), -65536, jnp.int32)
            for t in range(D_NODE // (2 * L)):
                acc_e = jnp.zeros((L,), jnp.float32)
                acc_o = jnp.zeros((L,), jnp.float32)
                for j in range(K):
                    vi = buf[i * K + j, pl.ds(t * L, L)]
                    ev = lax.bitcast_convert_type(
                        lax.shift_left(vi, 16), jnp.float32)
                    od = lax.bitcast_convert_type(vi & hi_mask, jnp.float32)
                    acc_e = acc_e + bjs[j] * ev
                    acc_o = acc_o + bjs[j] * od
                ost[i, pl.ds(t * L, L)] = acc_e
                ost[i, pl.ds(D_NODE // 2 + t * L, L)] = acc_o
            return carry
        lax.fori_loop(0, G, node_body, 0)

    def iter_body(it, carry):
        for r in range(NBUF):
            grp = it * NBUF + r
            p = r % 2
            pltpu.make_async_copy(
                shared.at[idx_v.at[grp]], bufs[r], sgs[r]).wait()

            @pl.when(grp >= 2)
            def _():
                pltpu.make_async_copy(
                    osts[p], out_hbm.at[pl.ds(base, G)], sos[p]).wait()

            compute(grp, bufs[r], osts[p])
            pltpu.async_copy(
                osts[p], out_hbm.at[pl.ds(base + grp * G, G)], sos[p])
            nxt = grp + NBUF

            @pl.when(nxt < NGRP)
            def _():
                pltpu.async_copy(shared.at[idx_v.at[nxt]], bufs[r], sgs[r])
        return carry

    lax.fori_loop(0, NGRP // NBUF, iter_body, 0)
    for p in range(2):
        pltpu.make_async_copy(
            osts[p], out_hbm.at[pl.ds(base, G)], sos[p]).wait()


def _sc_mix(nodes, nlist_w, b_w):
    mesh = plsc.VectorSubcoreMesh(core_axis_name="c", subcore_axis_name="s")
    kern = functools.partial(
        pl.kernel,
        mesh=mesh,
        out_type=jax.ShapeDtypeStruct((N_PAD, D_NODE), jnp.float32),
        scratch_types=[
            pltpu.VMEM((NGRP, ROWS), jnp.int32),
            pltpu.VMEM((PER_W * K,), jnp.float32),
        ] + [pltpu.VMEM((ROWS, D_NODE // 2), jnp.int32)] * NBUF
          + [pltpu.VMEM((G, D_NODE), jnp.float32)] * 2
          + [pltpu.VMEM_SHARED((N_TAB, D_NODE // 2), jnp.int32)]
          + [pltpu.SemaphoreType.DMA] * (NBUF + 2),
    )(_sc_mix_body)
    return kern(nodes, nlist_w, b_w)


def _out_body(mixed_ref, wv_ref, out_ref):
    out_ref[...] = jnp.maximum(
        jnp.dot(mixed_ref[...], wv_ref[...], preferred_element_type=jnp.float32),
        0.0)


def _out_proj(mixed_pad, wv):
    grid = N // BN
    return pl.pallas_call(
        _out_body,
        grid=(grid,),
        in_specs=[
            pl.BlockSpec((BN, D_NODE), lambda i: (i, 0)),
            pl.BlockSpec((D_NODE, D_NODE), lambda i: (0, 0)),
        ],
        out_specs=pl.BlockSpec((BN, D_NODE), lambda i: (i, 0)),
        out_shape=jax.ShapeDtypeStruct((N, D_NODE), jnp.float32),
    )(mixed_pad, wv)


def kernel(nodes, nlist, edges, inv_degree, wq, wk, wv):
    edges2d = edges.reshape(N, K * D_EDGE)
    inv2d = inv_degree.reshape(N, 1)

    b, nodes_pk = _attn_weights(nodes, edges2d, inv2d, wq, wk)

    nlist32 = nlist.astype(jnp.int32)
    nlist_pad = jnp.pad(nlist32, ((0, N_PAD - N), (0, 0)))
    nlist_w = nlist_pad.reshape(NW, NGRP, ROWS)
    b_pad = jnp.pad(b, ((0, N_PAD - N), (0, 0)))
    b_w = b_pad.reshape(NW, PER_W * K)

    mixed = _sc_mix(nodes_pk, nlist_w, b_w)
    return _out_proj(mixed, wv)


# speedup vs baseline: 1.0784x; 1.0046x over previous
import functools

import jax
import jax.numpy as jnp
from jax import lax
from jax.experimental import pallas as pl
from jax.experimental.pallas import tpu as pltpu
from jax.experimental.pallas import tpu_sc as plsc

N = 10000
K = 16
D_NODE = 256
D_EDGE = 16

L = 16
NC = 2
NS = 16
NW = NC * NS
G = 8
ROWS = G * K
N_PAD = 10240
PER_W = N_PAD // NW
NGRP = PER_W // G
BN = 2000


def _attn_body(nodes_ref, edges_ref, inv_ref, wq_ref, wk_ref, b_ref, pk_ref):
    f32 = jnp.float32
    q = jnp.dot(nodes_ref[...], wq_ref[...], preferred_element_type=f32)
    r = lax.dot_general(q, wk_ref[...], (((1,), (1,)), ((), ())),
                        preferred_element_type=f32)
    r = r * inv_ref[...]
    cmod = lax.broadcasted_iota(jnp.int32, (D_EDGE, K * D_EDGE), 1) % D_EDGE
    drow = lax.broadcasted_iota(jnp.int32, (D_EDGE, K * D_EDGE), 0)
    T = (cmod == drow).astype(f32)
    rt = jnp.dot(r, T, preferred_element_type=f32)
    p = rt * edges_ref[...]
    cdiv = lax.broadcasted_iota(jnp.int32, (K * D_EDGE, K), 0) // D_EDGE
    jcol = lax.broadcasted_iota(jnp.int32, (K * D_EDGE, K), 1)
    S = (cdiv == jcol).astype(f32)
    qd = jnp.dot(p, S, preferred_element_type=f32)
    m = jnp.max(qd, axis=1, keepdims=True)
    ex = jnp.exp(qd - m)
    b_ref[...] = ex / jnp.sum(ex, axis=1, keepdims=True)

    n = nodes_ref[...]

    def rne16(x):
        xb = lax.bitcast_convert_type(x, jnp.int32)
        xb = xb + 32767 + (lax.shift_right_logical(xb, 16) & 1)
        return lax.shift_right_logical(xb, 16)

    lo = rne16(n[:, :D_NODE // 2])
    hi = rne16(n[:, D_NODE // 2:])
    pk_ref[...] = lo | lax.shift_left(hi, 16)


def _attn_weights(nodes, edges2d, inv2d, wq, wk):
    grid = N // BN
    return pl.pallas_call(
        _attn_body,
        grid=(grid,),
        in_specs=[
            pl.BlockSpec((BN, D_NODE), lambda i: (i, 0)),
            pl.BlockSpec((BN, K * D_EDGE), lambda i: (i, 0)),
            pl.BlockSpec((BN, 1), lambda i: (i, 0)),
            pl.BlockSpec((D_NODE, D_EDGE), lambda i: (0, 0)),
            pl.BlockSpec((D_EDGE, D_EDGE), lambda i: (0, 0)),
        ],
        out_specs=[
            pl.BlockSpec((BN, K), lambda i: (i, 0)),
            pl.BlockSpec((BN, D_NODE // 2), lambda i: (i, 0)),
        ],
        out_shape=[
            jax.ShapeDtypeStruct((N, K), jnp.float32),
            jax.ShapeDtypeStruct((N_PAD, D_NODE // 2), jnp.int32),
        ],
    )(nodes, edges2d, inv2d, wq, wk)


NBUF = 2
N_TAB = 10112


def _sc_mix_body(nodes_hbm, nlist_hbm, b_hbm, out_hbm,
                 idx_v, b_v, buf0, buf1, ost0, ost1, shared,
                 sg0, sg1, so0, so1):
    bufs = (buf0, buf1)
    sgs = (sg0, sg1)
    osts = (ost0, ost1)
    sos = (so0, so1)

    wid = lax.axis_index("s") * NC + lax.axis_index("c")
    base = wid * PER_W

    s = lax.axis_index("s")
    chunk = N_TAB // NS
    pltpu.sync_copy(nodes_hbm.at[pl.ds(s * chunk, chunk)],
                    shared.at[pl.ds(s * chunk, chunk)])

    pltpu.sync_copy(nlist_hbm.at[wid], idx_v)
    pltpu.sync_copy(b_hbm.at[wid], b_v)
    plsc.subcore_barrier()

    for r in range(NBUF):
        pltpu.async_copy(shared.at[idx_v.at[r]], bufs[r], sgs[r])

    dnums = lax.GatherDimensionNumbers(
        offset_dims=(), collapsed_slice_dims=(0,), start_index_map=(0,))

    def compute(grp, buf, ost):
        def node_body(i, carry):
            node = grp * G + i
            bvec = b_v[pl.ds(node * K, K)]
            bjs = [
                lax.gather(
                    bvec, jnp.full((L, 1), j, jnp.int32), dnums,
                    slice_sizes=(1,),
                    mode=lax.GatherScatterMode.PROMISE_IN_BOUNDS)
                for j in range(K)
            ]
            hi_mask = jnp.full((L,), -65536, jnp.int32)
            for t in range(D_NODE // (2 * L)):
                acc_e = jnp.zeros((L,), jnp.float32)
                acc_o = jnp.zeros((L,), jnp.float32)
                for j in range(K):
                    vi = buf[i * K + j, pl.ds(t * L, L)]
                    ev = lax.bitcast_convert_type(
                        lax.shift_left(vi, 16), jnp.float32)
                    od = lax.bitcast_convert_type(vi & hi_mask, jnp.float32)
                    acc_e = acc_e + bjs[j] * ev
                    acc_o = acc_o + bjs[j] * od
                ost[i, pl.ds(t * L, L)] = acc_e
                ost[i, pl.ds(D_NODE // 2 + t * L, L)] = acc_o
            return carry
        lax.fori_loop(0, G, node_body, 0)

    def iter_body(it, carry):
        for r in range(NBUF):
            grp = it * NBUF + r
            p = r % 2
            pltpu.make_async_copy(
                shared.at[idx_v.at[grp]], bufs[r], sgs[r]).wait()

            @pl.when(grp >= 2)
            def _():
                pltpu.make_async_copy(
                    osts[p], out_hbm.at[pl.ds(base, G)], sos[p]).wait()

            compute(grp, bufs[r], osts[p])
            pltpu.async_copy(
                osts[p], out_hbm.at[pl.ds(base + grp * G, G)], sos[p])
            nxt = grp + NBUF

            @pl.when(nxt < NGRP)
            def _():
                pltpu.async_copy(shared.at[idx_v.at[nxt]], bufs[r], sgs[r])
        return carry

    lax.fori_loop(0, NGRP // NBUF, iter_body, 0)
    for p in range(2):
        pltpu.make_async_copy(
            osts[p], out_hbm.at[pl.ds(base, G)], sos[p]).wait()


def _sc_mix(nodes, nlist_w, b_w):
    mesh = plsc.VectorSubcoreMesh(core_axis_name="c", subcore_axis_name="s")
    kern = functools.partial(
        pl.kernel,
        mesh=mesh,
        out_type=jax.ShapeDtypeStruct((N_PAD, D_NODE), jnp.float32),
        scratch_types=[
            pltpu.VMEM((NGRP, ROWS), jnp.int32),
            pltpu.VMEM((PER_W * K,), jnp.float32),
        ] + [pltpu.VMEM((ROWS, D_NODE // 2), jnp.int32)] * NBUF
          + [pltpu.VMEM((G, D_NODE), jnp.float32)] * 2
          + [pltpu.VMEM_SHARED((N_TAB, D_NODE // 2), jnp.int32)]
          + [pltpu.SemaphoreType.DMA] * (NBUF + 2),
    )(_sc_mix_body)
    return kern(nodes, nlist_w, b_w)


def _out_body(mixed_ref, wv_ref, out_ref):
    out_ref[...] = jnp.maximum(
        jnp.dot(mixed_ref[...], wv_ref[...], preferred_element_type=jnp.float32),
        0.0)


def _out_proj(mixed_pad, wv):
    grid = N // BN
    return pl.pallas_call(
        _out_body,
        grid=(grid,),
        in_specs=[
            pl.BlockSpec((BN, D_NODE), lambda i: (i, 0)),
            pl.BlockSpec((D_NODE, D_NODE), lambda i: (0, 0)),
        ],
        out_specs=pl.BlockSpec((BN, D_NODE), lambda i: (i, 0)),
        out_shape=jax.ShapeDtypeStruct((N, D_NODE), jnp.float32),
    )(mixed_pad, wv)


def kernel(nodes, nlist, edges, inv_degree, wq, wk, wv):
    edges2d = edges.reshape(N, K * D_EDGE)
    inv2d = inv_degree.reshape(N, 1)

    b, nodes_pk = _attn_weights(nodes, edges2d, inv2d, wq, wk)

    nlist32 = nlist.astype(jnp.int32)
    nlist_pad = jnp.pad(nlist32, ((0, N_PAD - N), (0, 0)))
    nlist_w = nlist_pad.reshape(NW, NGRP, ROWS)
    b_pad = jnp.pad(b, ((0, N_PAD - N), (0, 0)))
    b_w = b_pad.reshape(NW, PER_W * K)

    mixed = _sc_mix(nodes_pk, nlist_w, b_w)
    return _out_proj(mixed, wv)
